# baseline (speedup 1.0000x reference)
import functools

import jax
import jax.numpy as jnp
from jax.experimental import pallas as pl
from jax.experimental.pallas import tpu as pltpu

E = 8
TOP_K = 2
D_IN = 1024
D_OUT = 1024
D_PROJ = 256
N_TOK = 4096

BM = 512
LANES = 128

_NEG = -1e30


def _gelu_tanh(x):
    return 0.5 * x * (1.0 + jnp.tanh(jnp.sqrt(2.0 / jnp.pi) * (x + 0.044715 * x ** 3)))


def _moe_kernel(x_ref, xb16_ref, wg_ref, bg_ref, w1_ref, b1_ref, w2_ref, b2_ref,
                out_ref, wfull_ref):
    e = pl.program_id(1)
    lane = jax.lax.broadcasted_iota(jnp.int32, (BM, LANES), 1)

    @pl.when(e == 0)
    def _gate():
        xb = x_ref[...]
        logits = (jnp.dot(xb, wg_ref[...], preferred_element_type=jnp.float32)
                  + bg_ref[...]) * (1.0 / jnp.sqrt(jnp.float32(D_IN)))
        logits = jnp.where(lane < E, logits, _NEG)
        m1 = jnp.max(logits, axis=1, keepdims=True)
        p = jnp.exp(logits - m1)
        probs = p / jnp.sum(p, axis=1, keepdims=True)
        i1 = jnp.min(jnp.where(logits >= m1, lane, LANES), axis=1, keepdims=True)
        logits2 = jnp.where(lane == i1, _NEG, logits)
        m2 = jnp.max(logits2, axis=1, keepdims=True)
        i2 = jnp.min(jnp.where(logits2 >= m2, lane, LANES), axis=1, keepdims=True)
        wfull_ref[...] = probs * ((lane == i1) | (lane == i2)).astype(jnp.float32)

    w_col = jnp.sum(
        wfull_ref[...] * (lane == e).astype(jnp.float32), axis=1, keepdims=True)
    h = _gelu_tanh(
        jnp.dot(xb16_ref[...], w1_ref[0], preferred_element_type=jnp.float32)
        + b1_ref[0])
    y = (jnp.dot(h.astype(jnp.bfloat16), w2_ref[0],
                 preferred_element_type=jnp.float32) + b2_ref[0])
    contrib = w_col * y

    @pl.when(e == 0)
    def _init():
        out_ref[...] = contrib

    @pl.when(e != 0)
    def _acc():
        out_ref[...] += contrib


@jax.jit
def kernel(x, Wg, bg, W1, b1, W2, b2):
    in_shape = x.shape
    xf = x.reshape(-1, D_IN)
    n = xf.shape[0]
    wg_pad = jnp.pad(Wg, ((0, 0), (0, LANES - E)))
    bg_pad = jnp.pad(bg, (0, LANES - E)).reshape(1, LANES)
    b1r = b1.reshape(E, 1, D_PROJ)
    b2r = b2.reshape(E, 1, D_OUT)
    x16 = xf.astype(jnp.bfloat16)
    W1_16 = W1.astype(jnp.bfloat16)
    W2_16 = W2.astype(jnp.bfloat16)
    grid = (n // BM, E)
    y = pl.pallas_call(
        _moe_kernel,
        grid=grid,
        in_specs=[
            pl.BlockSpec((BM, D_IN), lambda i, e: (i, 0)),
            pl.BlockSpec((BM, D_IN), lambda i, e: (i, 0)),
            pl.BlockSpec((D_IN, LANES), lambda i, e: (0, 0)),
            pl.BlockSpec((1, LANES), lambda i, e: (0, 0)),
            pl.BlockSpec((1, D_IN, D_PROJ), lambda i, e: (e, 0, 0)),
            pl.BlockSpec((1, 1, D_PROJ), lambda i, e: (e, 0, 0)),
            pl.BlockSpec((1, D_PROJ, D_OUT), lambda i, e: (e, 0, 0)),
            pl.BlockSpec((1, 1, D_OUT), lambda i, e: (e, 0, 0)),
        ],
        out_specs=pl.BlockSpec((BM, D_OUT), lambda i, e: (i, 0)),
        out_shape=jax.ShapeDtypeStruct((n, D_OUT), jnp.float32),
        scratch_shapes=[pltpu.VMEM((BM, LANES), jnp.float32)],
        compiler_params=pltpu.CompilerParams(
            dimension_semantics=("parallel", "arbitrary")),
    )(xf, x16, wg_pad, bg_pad, W1_16, b1r, W2_16, b2r)
    return y.reshape(in_shape[:-1] + (D_OUT,))


# speedup vs baseline: 1.3880x; 1.3880x over previous
import functools

import jax
import jax.numpy as jnp
from jax.experimental import pallas as pl
from jax.experimental.pallas import tpu as pltpu

E = 8
TOP_K = 2
D_IN = 1024
D_OUT = 1024
D_PROJ = 256
N_TOK = 4096

BM = 512
LANES = 128
D_CAT = E * D_PROJ

_NEG = -1e30


def _gelu_tanh(x):
    return 0.5 * x * (1.0 + jnp.tanh(jnp.sqrt(2.0 / jnp.pi) * (x + 0.044715 * x ** 3)))


def _moe_kernel(x_ref, xb16_ref, wg_ref, bg_ref, w1_ref, b1_ref, w2_ref, b2_ref,
                out_ref):
    lane = jax.lax.broadcasted_iota(jnp.int32, (BM, LANES), 1)

    logits = (jnp.dot(x_ref[...], wg_ref[...], preferred_element_type=jnp.float32)
              + bg_ref[...]) * (1.0 / jnp.sqrt(jnp.float32(D_IN)))
    logits = jnp.where(lane < E, logits, _NEG)
    m1 = jnp.max(logits, axis=1, keepdims=True)
    p = jnp.exp(logits - m1)
    probs = p / jnp.sum(p, axis=1, keepdims=True)
    i1 = jnp.min(jnp.where(logits >= m1, lane, LANES), axis=1, keepdims=True)
    logits2 = jnp.where(lane == i1, _NEG, logits)
    m2 = jnp.max(logits2, axis=1, keepdims=True)
    i2 = jnp.min(jnp.where(logits2 >= m2, lane, LANES), axis=1, keepdims=True)
    wfull = probs * ((lane == i1) | (lane == i2)).astype(jnp.float32)

    h = _gelu_tanh(
        jnp.dot(xb16_ref[...], w1_ref[...], preferred_element_type=jnp.float32)
        + b1_ref[...])
    cols = []
    for g in range(E):
        cols.append(h[:, g * D_PROJ:(g + 1) * D_PROJ] * wfull[:, g:g + 1])
    g16 = jnp.concatenate(cols, axis=1).astype(jnp.bfloat16)
    y = jnp.dot(g16, w2_ref[...], preferred_element_type=jnp.float32)
    y += jnp.dot(wfull.astype(jnp.bfloat16), b2_ref[...],
                 preferred_element_type=jnp.float32)
    out_ref[...] = y


@jax.jit
def kernel(x, Wg, bg, W1, b1, W2, b2):
    in_shape = x.shape
    xf = x.reshape(-1, D_IN)
    n = xf.shape[0]
    wg_pad = jnp.pad(Wg, ((0, 0), (0, LANES - E)))
    bg_pad = jnp.pad(bg, (0, LANES - E)).reshape(1, LANES)
    w1_cat = W1.transpose(1, 0, 2).reshape(D_IN, D_CAT).astype(jnp.bfloat16)
    b1_cat = b1.reshape(1, D_CAT)
    w2_stack = W2.reshape(D_CAT, D_OUT).astype(jnp.bfloat16)
    b2_pad = jnp.pad(b2, ((0, LANES - E), (0, 0))).astype(jnp.bfloat16)
    x16 = xf.astype(jnp.bfloat16)
    grid = (n // BM,)
    y = pl.pallas_call(
        _moe_kernel,
        grid=grid,
        in_specs=[
            pl.BlockSpec((BM, D_IN), lambda i: (i, 0)),
            pl.BlockSpec((BM, D_IN), lambda i: (i, 0)),
            pl.BlockSpec((D_IN, LANES), lambda i: (0, 0)),
            pl.BlockSpec((1, LANES), lambda i: (0, 0)),
            pl.BlockSpec((D_IN, D_CAT), lambda i: (0, 0)),
            pl.BlockSpec((1, D_CAT), lambda i: (0, 0)),
            pl.BlockSpec((D_CAT, D_OUT), lambda i: (0, 0)),
            pl.BlockSpec((LANES, D_OUT), lambda i: (0, 0)),
        ],
        out_specs=pl.BlockSpec((BM, D_OUT), lambda i: (i, 0)),
        out_shape=jax.ShapeDtypeStruct((n, D_OUT), jnp.float32),
        compiler_params=pltpu.CompilerParams(
            dimension_semantics=("parallel",)),
    )(xf, x16, wg_pad, bg_pad, w1_cat, b1_cat, w2_stack, b2_pad)
    return y.reshape(in_shape[:-1] + (D_OUT,))


# speedup vs baseline: 1.5605x; 1.1243x over previous
import functools

import jax
import jax.numpy as jnp
from jax.experimental import pallas as pl
from jax.experimental.pallas import tpu as pltpu

E = 8
TOP_K = 2
D_IN = 1024
D_OUT = 1024
D_PROJ = 256
N_TOK = 4096

BM = 512
LANES = 128
D_CAT = E * D_PROJ

_NEG = -1e30


def _gelu_tanh(x):
    return 0.5 * x * (1.0 + jnp.tanh(jnp.sqrt(2.0 / jnp.pi) * (x + 0.044715 * x ** 3)))


def _moe_kernel(x_ref, wg_ref, bg_ref, w1_ref, b1_ref, w2_ref, b2_ref,
                out_ref):
    lane = jax.lax.broadcasted_iota(jnp.int32, (BM, LANES), 1)
    xb = x_ref[...]

    logits = (jnp.dot(xb, wg_ref[...], preferred_element_type=jnp.float32)
              + bg_ref[...]) * (1.0 / jnp.sqrt(jnp.float32(D_IN)))
    logits = jnp.where(lane < E, logits, _NEG)
    m1 = jnp.max(logits, axis=1, keepdims=True)
    p = jnp.exp(logits - m1)
    probs = p / jnp.sum(p, axis=1, keepdims=True)
    i1 = jnp.min(jnp.where(logits >= m1, lane, LANES), axis=1, keepdims=True)
    logits2 = jnp.where(lane == i1, _NEG, logits)
    m2 = jnp.max(logits2, axis=1, keepdims=True)
    i2 = jnp.min(jnp.where(logits2 >= m2, lane, LANES), axis=1, keepdims=True)
    wfull = probs * ((lane == i1) | (lane == i2)).astype(jnp.float32)

    xb16 = xb.astype(jnp.bfloat16)
    cols = []
    for g in range(E):
        hg = (jnp.dot(xb16, w1_ref[g], preferred_element_type=jnp.float32)
              + b1_ref[:, g * D_PROJ:(g + 1) * D_PROJ])
        cols.append((_gelu_tanh(hg) * wfull[:, g:g + 1]).astype(jnp.bfloat16))
    g16 = jnp.concatenate(cols, axis=1)
    y = jnp.dot(g16, w2_ref[...], preferred_element_type=jnp.float32)
    y += jnp.dot(wfull.astype(jnp.bfloat16), b2_ref[...],
                 preferred_element_type=jnp.float32)
    out_ref[...] = y


@jax.jit
def kernel(x, Wg, bg, W1, b1, W2, b2):
    in_shape = x.shape
    xf = x.reshape(-1, D_IN)
    n = xf.shape[0]
    wg_pad = jnp.pad(Wg, ((0, 0), (0, LANES - E)))
    bg_pad = jnp.pad(bg, (0, LANES - E)).reshape(1, LANES)
    w1_16 = W1.astype(jnp.bfloat16)
    b1_cat = b1.reshape(1, D_CAT)
    w2_stack = W2.astype(jnp.bfloat16).reshape(D_CAT, D_OUT)
    b2_pad = jnp.pad(b2, ((0, LANES - E), (0, 0))).astype(jnp.bfloat16)
    grid = (n // BM,)
    y = pl.pallas_call(
        _moe_kernel,
        grid=grid,
        in_specs=[
            pl.BlockSpec((BM, D_IN), lambda i: (i, 0)),
            pl.BlockSpec((D_IN, LANES), lambda i: (0, 0)),
            pl.BlockSpec((1, LANES), lambda i: (0, 0)),
            pl.BlockSpec((E, D_IN, D_PROJ), lambda i: (0, 0, 0)),
            pl.BlockSpec((1, D_CAT), lambda i: (0, 0)),
            pl.BlockSpec((D_CAT, D_OUT), lambda i: (0, 0)),
            pl.BlockSpec((LANES, D_OUT), lambda i: (0, 0)),
        ],
        out_specs=pl.BlockSpec((BM, D_OUT), lambda i: (i, 0)),
        out_shape=jax.ShapeDtypeStruct((n, D_OUT), jnp.float32),
        compiler_params=pltpu.CompilerParams(
            dimension_semantics=("parallel",)),
    )(xf, wg_pad, bg_pad, w1_16, b1_cat, w2_stack, b2_pad)
    return y.reshape(in_shape[:-1] + (D_OUT,))


# speedup vs baseline: 1.9978x; 1.2802x over previous
import functools

import jax
import jax.numpy as jnp
from jax.experimental import pallas as pl
from jax.experimental.pallas import tpu as pltpu

E = 8
TOP_K = 2
D_IN = 1024
D_OUT = 1024
D_PROJ = 256
N_TOK = 4096

BM = 1024
CH = 256
LANES = 128
D_CAT = E * D_PROJ

_NEG = -1e30


def _gelu_tanh(x):
    return 0.5 * x * (1.0 + jnp.tanh(jnp.sqrt(2.0 / jnp.pi) * (x + 0.044715 * x ** 3)))


def _moe_kernel(x_ref, wg_ref, bg_ref, w1_ref, b1_ref, w2_ref, b2_ref,
                out_ref):
    lane = jax.lax.broadcasted_iota(jnp.int32, (CH, LANES), 1)
    for c in range(BM // CH):
        rows = pl.ds(c * CH, CH)
        xb = x_ref[rows, :]

        logits = (jnp.dot(xb, wg_ref[...], preferred_element_type=jnp.float32)
                  + bg_ref[...]) * (1.0 / jnp.sqrt(jnp.float32(D_IN)))
        logits = jnp.where(lane < E, logits, _NEG)
        m1 = jnp.max(logits, axis=1, keepdims=True)
        p = jnp.exp(logits - m1)
        probs = p / jnp.sum(p, axis=1, keepdims=True)
        i1 = jnp.min(jnp.where(logits >= m1, lane, LANES), axis=1, keepdims=True)
        logits2 = jnp.where(lane == i1, _NEG, logits)
        m2 = jnp.max(logits2, axis=1, keepdims=True)
        i2 = jnp.min(jnp.where(logits2 >= m2, lane, LANES), axis=1, keepdims=True)
        wfull = probs * ((lane == i1) | (lane == i2)).astype(jnp.float32)

        xb16 = xb.astype(jnp.bfloat16)
        cols = []
        for g in range(E):
            hg = (jnp.dot(xb16, w1_ref[g], preferred_element_type=jnp.float32)
                  + b1_ref[:, g * D_PROJ:(g + 1) * D_PROJ])
            cols.append((_gelu_tanh(hg) * wfull[:, g:g + 1]).astype(jnp.bfloat16))
        g16 = jnp.concatenate(cols, axis=1)
        y = jnp.dot(g16, w2_ref[...], preferred_element_type=jnp.float32)
        y += jnp.dot(wfull.astype(jnp.bfloat16), b2_ref[...],
                     preferred_element_type=jnp.float32)
        out_ref[rows, :] = y


@jax.jit
def kernel(x, Wg, bg, W1, b1, W2, b2):
    in_shape = x.shape
    xf = x.reshape(-1, D_IN)
    n = xf.shape[0]
    wg_pad = jnp.pad(Wg, ((0, 0), (0, LANES - E)))
    bg_pad = jnp.pad(bg, (0, LANES - E)).reshape(1, LANES)
    w1_16 = W1.astype(jnp.bfloat16)
    b1_cat = b1.reshape(1, D_CAT)
    w2_stack = W2.astype(jnp.bfloat16).reshape(D_CAT, D_OUT)
    b2_pad = jnp.pad(b2, ((0, LANES - E), (0, 0))).astype(jnp.bfloat16)
    grid = (n // BM,)
    y = pl.pallas_call(
        _moe_kernel,
        grid=grid,
        in_specs=[
            pl.BlockSpec((BM, D_IN), lambda i: (i, 0)),
            pl.BlockSpec((D_IN, LANES), lambda i: (0, 0)),
            pl.BlockSpec((1, LANES), lambda i: (0, 0)),
            pl.BlockSpec((E, D_IN, D_PROJ), lambda i: (0, 0, 0)),
            pl.BlockSpec((1, D_CAT), lambda i: (0, 0)),
            pl.BlockSpec((D_CAT, D_OUT), lambda i: (0, 0)),
            pl.BlockSpec((LANES, D_OUT), lambda i: (0, 0)),
        ],
        out_specs=pl.BlockSpec((BM, D_OUT), lambda i: (i, 0)),
        out_shape=jax.ShapeDtypeStruct((n, D_OUT), jnp.float32),
        compiler_params=pltpu.CompilerParams(
            dimension_semantics=("parallel",)),
    )(xf, wg_pad, bg_pad, w1_16, b1_cat, w2_stack, b2_pad)
    return y.reshape(in_shape[:-1] + (D_OUT,))


# speedup vs baseline: 2.2490x; 1.1258x over previous
import functools

import jax
import jax.numpy as jnp
from jax.experimental import pallas as pl
from jax.experimental.pallas import tpu as pltpu

E = 8
TOP_K = 2
D_IN = 1024
D_OUT = 1024
D_PROJ = 256
N_TOK = 4096

BM = 1024
CH = 256
LANES = 128
D_CAT = E * D_PROJ

_NEG = -1e30


def _gelu_tanh(x):
    return 0.5 * x * (1.0 + jnp.tanh(jnp.sqrt(2.0 / jnp.pi) * (x + 0.044715 * x ** 3)))


def _moe_kernel(x_ref, wg_ref, bg_ref, w1_ref, b1_ref, w2_ref, b2_ref,
                out_ref):
    lane = jax.lax.broadcasted_iota(jnp.int32, (CH, LANES), 1)
    for c in range(BM // CH):
        rows = pl.ds(c * CH, CH)
        xb = x_ref[rows, :]

        logits = (jnp.dot(xb, wg_ref[...], preferred_element_type=jnp.float32)
                  + bg_ref[...]) * (1.0 / jnp.sqrt(jnp.float32(D_IN)))
        logits = jnp.where(lane < E, logits, _NEG)
        m1 = jnp.max(logits, axis=1, keepdims=True)
        p = jnp.exp(logits - m1)
        probs = p / jnp.sum(p, axis=1, keepdims=True)
        i1 = jnp.min(jnp.where(logits >= m1, lane, LANES), axis=1, keepdims=True)
        logits2 = jnp.where(lane == i1, _NEG, logits)
        m2 = jnp.max(logits2, axis=1, keepdims=True)
        i2 = jnp.min(jnp.where(logits2 >= m2, lane, LANES), axis=1, keepdims=True)
        wfull = probs * ((lane == i1) | (lane == i2)).astype(jnp.float32)

        xb16 = xb.astype(jnp.bfloat16)
        cols = []
        for g in range(E):
            w1g = w1_ref[g].astype(jnp.bfloat16)
            hg = (jnp.dot(xb16, w1g, preferred_element_type=jnp.float32)
                  + b1_ref[:, g * D_PROJ:(g + 1) * D_PROJ])
            cols.append((_gelu_tanh(hg) * wfull[:, g:g + 1]).astype(jnp.bfloat16))
        g16 = jnp.concatenate(cols, axis=1)
        y = jnp.dot(g16, w2_ref[...].astype(jnp.bfloat16),
                    preferred_element_type=jnp.float32)
        y += jnp.dot(wfull, b2_ref[...], preferred_element_type=jnp.float32)
        out_ref[rows, :] = y


@jax.jit
def kernel(x, Wg, bg, W1, b1, W2, b2):
    in_shape = x.shape
    xf = x.reshape(-1, D_IN)
    n = xf.shape[0]
    wg_pad = jnp.pad(Wg, ((0, 0), (0, LANES - E)))
    bg_pad = jnp.pad(bg, (0, LANES - E)).reshape(1, LANES)
    b1_cat = b1.reshape(1, D_CAT)
    w2_stack = W2.reshape(D_CAT, D_OUT)
    b2_pad = jnp.pad(b2, ((0, LANES - E), (0, 0)))
    grid = (n // BM,)
    y = pl.pallas_call(
        _moe_kernel,
        grid=grid,
        in_specs=[
            pl.BlockSpec((BM, D_IN), lambda i: (i, 0)),
            pl.BlockSpec((D_IN, LANES), lambda i: (0, 0)),
            pl.BlockSpec((1, LANES), lambda i: (0, 0)),
            pl.BlockSpec((E, D_IN, D_PROJ), lambda i: (0, 0, 0)),
            pl.BlockSpec((1, D_CAT), lambda i: (0, 0)),
            pl.BlockSpec((D_CAT, D_OUT), lambda i: (0, 0)),
            pl.BlockSpec((LANES, D_OUT), lambda i: (0, 0)),
        ],
        out_specs=pl.BlockSpec((BM, D_OUT), lambda i: (i, 0)),
        out_shape=jax.ShapeDtypeStruct((n, D_OUT), jnp.float32),
        compiler_params=pltpu.CompilerParams(
            dimension_semantics=("parallel",)),
    )(xf, wg_pad, bg_pad, W1, b1_cat, w2_stack, b2_pad)
    return y.reshape(in_shape[:-1] + (D_OUT,))
